# baseline (speedup 1.0000x reference)
import jax
import jax.numpy as jnp
from jax.experimental import pallas as pl
from jax.experimental.pallas import tpu as pltpu

_K = 4
_QS = 1250
_QP = 1280


def _body(ts_ref, lg0, lg1, lg2, lg3, bx_ref, cls_ref,
          scores_ref, labels_ref, boxes_out_ref, cls_out_ref):
    i = pl.program_id(0)
    zf = jnp.zeros((_QP - _QS,), jnp.float32)
    zi = jnp.zeros((_QP - _QS,), jnp.int32)

    for k, lg in enumerate((lg0, lg1, lg2, lg3)):
        xt = lg[0, 0].T
        c_iota = jax.lax.broadcasted_iota(jnp.int32, xt.shape, 0)
        m = jnp.max(xt, axis=0)
        labels = jnp.min(jnp.where(xt == m[None, :], c_iota, 91), axis=0)
        s = jnp.sum(jnp.exp(xt), axis=0)
        scores_ref[0, k] = jnp.concatenate([jnp.exp(m) / s, zf])
        labels_ref[0, k] = jnp.concatenate([labels, zi])

    sh = ts_ref[i, 0].astype(jnp.float32)
    sw = ts_ref[i, 1].astype(jnp.float32)
    for k in range(_K):
        bt = bx_ref[0, k].T
        cx, cy, w, h = bt[0], bt[1], bt[2], bt[3]
        x0 = jnp.clip(cx - 0.5 * w, 0.0, 1.0) * sw
        y0 = jnp.clip(cy - 0.5 * h, 0.0, 1.0) * sh
        x1 = jnp.clip(cx + 0.5 * w, 0.0, 1.0) * sw
        y1 = jnp.clip(cy + 0.5 * h, 0.0, 1.0) * sh
        boxes_out_ref[0, k] = jnp.stack([x0, y0, x1, y1], axis=0).T

    @pl.when(i == 0)
    def _():
        c = cls_ref[...]
        cm = jnp.max(c, axis=-1, keepdims=True)
        ci = jax.lax.broadcasted_iota(jnp.int32, c.shape, 1)
        cls_out_ref[...] = jnp.min(jnp.where(c == cm, ci, 10),
                                   axis=-1, keepdims=True)


def kernel(pred_logits, pred_boxes, cls_logits, target_sizes):
    nb, nq, nc = pred_logits.shape
    lg = pred_logits.reshape(nb, _K, _QS, nc)
    bx = pred_boxes.reshape(nb, _K, _QS, 4)
    lspec = [pl.BlockSpec((1, 1, _QS, nc), lambda i, k=k: (i, k, 0, 0))
             for k in range(_K)]
    scores, labels, boxes, cls2 = pl.pallas_call(
        _body,
        grid=(nb,),
        in_specs=[pl.BlockSpec(memory_space=pltpu.SMEM)] + lspec + [
            pl.BlockSpec((1, _K, _QS, 4), lambda i: (i, 0, 0, 0)),
            pl.BlockSpec((16, 10), lambda i: (0, 0)),
        ],
        out_specs=[
            pl.BlockSpec((1, _K, _QP), lambda i: (i, 0, 0)),
            pl.BlockSpec((1, _K, _QP), lambda i: (i, 0, 0)),
            pl.BlockSpec((1, _K, _QS, 4), lambda i: (i, 0, 0, 0)),
            pl.BlockSpec((16, 1), lambda i: (0, 0)),
        ],
        out_shape=[
            jax.ShapeDtypeStruct((nb, _K, _QP), jnp.float32),
            jax.ShapeDtypeStruct((nb, _K, _QP), jnp.int32),
            jax.ShapeDtypeStruct((nb, _K, _QS, 4), jnp.float32),
            jax.ShapeDtypeStruct((16, 1), jnp.int32),
        ],
    )(target_sizes, *(lg,) * _K, bx, cls_logits)
    return (scores[:, :, :_QS].reshape(nb, nq),
            labels[:, :, :_QS].reshape(nb, nq),
            boxes.reshape(nb, nq, 4), cls2.reshape(nb))
